# baseline (speedup 1.0000x reference)
import functools

import jax
import jax.numpy as jnp
from jax import lax
from jax.experimental import pallas as pl
from jax.experimental.pallas import tpu as pltpu
from jax.experimental.pallas import tpu_sc as plsc

N, E, D, H = 10000, 320000, 128, 128
Np = 10240
NSC, NTILE = 2, 16
SLABS = NSC * NTILE
CH, B = 88, 128
KB = 8
NB = CH // KB
WIN = 2048
WSH = 11
NBK = Np // WIN
CAPC = CH
CAPW = NBK * CAPC * B
SCAP = CAPC * B + 128
L = SLABS * CH * B
RPT = Np // NTILE

_MESH = plsc.VectorSubcoreMesh(
    core_axis_name="c", subcore_axis_name="s",
    num_cores=NSC, num_subcores=NTILE)


def _deg_body(col_hbm, ew_hbm, out_hbm, colv, ewv, bounce, acc):
    c = lax.axis_index("c")
    s = lax.axis_index("s")
    w = c * NTILE + s

    def zb(i, carry):
        bounce[pl.ds(i * 16, 16)] = jnp.zeros((16,), jnp.float32)
        return carry
    lax.fori_loop(0, RPT // 16, zb, 0)
    pltpu.sync_copy(bounce, acc.at[pl.ds(s * RPT, RPT)])
    plsc.subcore_barrier()

    def blk_body(blk, carry):
        pltpu.sync_copy(col_hbm.at[w, pl.ds(blk * KB, KB)], colv)
        pltpu.sync_copy(ew_hbm.at[w, pl.ds(blk * KB, KB)], ewv)

        def body(i, carry2):
            pltpu.sync_copy(ewv.at[i], acc.at[colv.at[i]], add=True)
            return carry2
        lax.fori_loop(0, KB, body, 0)
        return carry
    lax.fori_loop(0, NB, blk_body, 0)
    plsc.subcore_barrier()

    pltpu.sync_copy(acc.at[pl.ds(s * RPT, RPT)], bounce)
    pltpu.sync_copy(bounce, out_hbm.at[c, pl.ds(s * RPT, RPT)])


_deg = functools.partial(
    pl.kernel,
    out_type=jax.ShapeDtypeStruct((NSC, Np), jnp.float32),
    mesh=_MESH,
    compiler_params=pltpu.CompilerParams(needs_layout_passes=False),
    scratch_types=[
        pltpu.VMEM((KB, B), jnp.int32),
        pltpu.VMEM((KB, B), jnp.float32),
        pltpu.VMEM((RPT,), jnp.float32),
        pltpu.VMEM_SHARED((Np,), jnp.float32),
    ],
)(_deg_body)


def _part_body(row_hbm, col_hbm, ew_hbm, dis_hbm,
               prow_hbm, pcol_hbm, pnorm_hbm, cnt_hbm,
               rowv, colv, ewv, srow, scol, snorm, disv, cntv):
    c = lax.axis_index("c")
    s = lax.axis_index("s")
    w = c * NTILE + s
    pltpu.sync_copy(dis_hbm, disv)

    zi = jnp.zeros((16,), jnp.int32)
    zc = jnp.full((16,), Np - 1, jnp.int32)
    zf = jnp.zeros((16,), jnp.float32)

    def prefill(n16, _):
        def pf(i, carry):
            sl = pl.ds(i * 16, 16)
            srow[sl] = zi
            scol[sl] = zc
            snorm[sl] = zf
            return carry
        lax.fori_loop(0, n16, pf, 0)

    prefill(SCAP // 16, None)
    nbs = []
    for b in range(NBK):
        def blk(kb, off):
            sl_blk = pl.ds(kb * KB, KB)
            pltpu.sync_copy(row_hbm.at[w, sl_blk], rowv)
            pltpu.sync_copy(col_hbm.at[w, sl_blk], colv)
            pltpu.sync_copy(ew_hbm.at[w, sl_blk], ewv)

            def ch(i, off2):
                for j in range(B // 16):
                    sl = pl.ds(j * 16, 16)
                    r16 = rowv[i, sl]
                    c16 = colv[i, sl]
                    e16 = ewv[i, sl]
                    dr = plsc.load_gather(disv, [r16])
                    dc = plsc.load_gather(disv, [c16])
                    nrm = dr * e16 * dc
                    rel = r16 - b * WIN
                    msk = lax.shift_right_logical(r16, WSH) == b
                    plsc.store_compressed(srow.at[pl.ds(off2, 16)], rel, mask=msk)
                    plsc.store_compressed(scol.at[pl.ds(off2, 16)], c16, mask=msk)
                    plsc.store_compressed(snorm.at[pl.ds(off2, 16)], nrm, mask=msk)
                    pc = plsc.all_reduce_population_count(msk)
                    off2 = off2 + pc[0]
                return off2
            return lax.fori_loop(0, KB, ch, off)
        off = lax.fori_loop(0, NB, blk, 0)
        fl = pl.ds(b * CAPC * B, CAPC * B)
        pltpu.sync_copy(srow.at[pl.ds(0, CAPC * B)], prow_hbm.at[w, fl])
        pltpu.sync_copy(scol.at[pl.ds(0, CAPC * B)], pcol_hbm.at[w, fl])
        pltpu.sync_copy(snorm.at[pl.ds(0, CAPC * B)], pnorm_hbm.at[w, fl])
        nbs.append((off + B - 1) // B)
        prefill((off + 15) // 16, None)

    lanes = lax.iota(jnp.int32, 16)
    cv = jnp.zeros((16,), jnp.int32)
    for b in range(NBK):
        cv = jnp.where(lanes == b, jnp.full((16,), nbs[b]), cv)
    z16 = jnp.zeros((16,), jnp.int32)
    for q in range(8):
        cntv[pl.ds(q * 16, 16)] = z16
    cntv[pl.ds(0, 16)] = cv
    pltpu.sync_copy(cntv, cnt_hbm.at[w])


_part = functools.partial(
    pl.kernel,
    out_type=(jax.ShapeDtypeStruct((SLABS, CAPW), jnp.int32),
              jax.ShapeDtypeStruct((SLABS, CAPW), jnp.int32),
              jax.ShapeDtypeStruct((SLABS, CAPW), jnp.float32),
              jax.ShapeDtypeStruct((SLABS, 128), jnp.int32)),
    mesh=_MESH,
    compiler_params=pltpu.CompilerParams(needs_layout_passes=False),
    scratch_types=[
        pltpu.VMEM((KB, B), jnp.int32),
        pltpu.VMEM((KB, B), jnp.int32),
        pltpu.VMEM((KB, B), jnp.float32),
        pltpu.VMEM((SCAP,), jnp.int32),
        pltpu.VMEM((SCAP,), jnp.int32),
        pltpu.VMEM((SCAP,), jnp.float32),
        pltpu.VMEM((Np,), jnp.float32),
        pltpu.VMEM((128,), jnp.int32),
    ],
)(_part_body)


def _agg_body(h_hbm, prow_hbm, pcol_hbm, pnorm_hbm, cnt_hbm, out_hbm,
              rowf, colf, normf, rowv2, colv2, buf, cntv, win, acc):
    c = lax.axis_index("c")
    s = lax.axis_index("s")
    w = c * NTILE + s
    pltpu.sync_copy(cnt_hbm.at[w], cntv)

    def zb(r, carry):
        for j in range(B // 16):
            buf[r, pl.ds(j * 16, 16)] = jnp.zeros((16,), jnp.float32)
        return carry
    lax.fori_loop(0, B, zb, 0)
    for k in range(RPT // B):
        pltpu.sync_copy(buf, acc.at[pl.ds(s * RPT + k * B, B)])

    for b in range(NBK):
        plsc.subcore_barrier()
        pltpu.sync_copy(h_hbm.at[pl.ds(b * WIN + s * B, B)],
                        win.at[pl.ds(s * B, B)])
        plsc.subcore_barrier()

        cv = cntv[pl.ds(0, 16)]
        nch = cv[b]
        nblk = (nch + KB - 1) // KB

        def blk(kb, carry):
            off = (b * CAPC + kb * KB) * B
            pltpu.sync_copy(prow_hbm.at[w, pl.ds(off, KB * B)], rowf)
            pltpu.sync_copy(pcol_hbm.at[w, pl.ds(off, KB * B)], colf)
            pltpu.sync_copy(pnorm_hbm.at[w, pl.ds(off, KB * B)], normf)

            def rsh(j, carry2):
                for k2 in range(B // 16):
                    sl = pl.ds(k2 * 16, 16)
                    fsl = pl.ds(j * B + k2 * 16, 16)
                    rowv2[j, sl] = rowf[fsl]
                    colv2[j, sl] = colf[fsl]
                return carry2
            lax.fori_loop(0, KB, rsh, 0)

            nin = jnp.minimum(nch - kb * KB, KB)

            def chunk(i, carry2):
                pltpu.sync_copy(win.at[rowv2.at[i]], buf)

                def scale(rg, carry3):
                    nv16 = normf[pl.ds(i * B + rg * 16, 16)]
                    for r in range(16):
                        nv = lax.gather(
                            nv16, jnp.full((16, 1), r, jnp.int32),
                            lax.GatherDimensionNumbers(
                                offset_dims=(), collapsed_slice_dims=(0,),
                                start_index_map=(0,)),
                            (1,), mode=lax.GatherScatterMode.PROMISE_IN_BOUNDS)
                        row = rg * 16 + r
                        for j in range(B // 16):
                            sl = pl.ds(j * 16, 16)
                            buf[row, sl] = buf[row, sl] * nv
                    return carry3
                lax.fori_loop(0, B // 16, scale, 0)
                pltpu.sync_copy(buf, acc.at[colv2.at[i]], add=True)
                return carry2
            lax.fori_loop(0, nin, chunk, 0)
            return carry
        lax.fori_loop(0, nblk, blk, 0)
    plsc.subcore_barrier()

    for k in range(RPT // B):
        rs = s * RPT + k * B
        pltpu.sync_copy(acc.at[pl.ds(rs, B)], buf)
        pltpu.sync_copy(buf, out_hbm.at[c, pl.ds(rs, B)])


_agg = functools.partial(
    pl.kernel,
    out_type=jax.ShapeDtypeStruct((NSC, Np, H), jnp.float32),
    mesh=_MESH,
    compiler_params=pltpu.CompilerParams(needs_layout_passes=False),
    scratch_types=[
        pltpu.VMEM((KB * B,), jnp.int32),
        pltpu.VMEM((KB * B,), jnp.int32),
        pltpu.VMEM((KB * B,), jnp.float32),
        pltpu.VMEM((KB, B), jnp.int32),
        pltpu.VMEM((KB, B), jnp.int32),
        pltpu.VMEM((B, H), jnp.float32),
        pltpu.VMEM((128,), jnp.int32),
        pltpu.VMEM_SHARED((WIN, H), jnp.float32),
        pltpu.VMEM_SHARED((Np, H), jnp.float32),
    ],
)(_agg_body)


_BM = 512


def _dis_body(pd_ref, o_ref):
    deg = pd_ref[0] + pd_ref[1]
    o_ref[...] = lax.rsqrt(jnp.maximum(deg, 1e-12))


def _dis(pdeg):
    pd = pdeg.reshape(NSC, Np // 128, 128)
    out = pl.pallas_call(
        _dis_body,
        out_shape=jax.ShapeDtypeStruct((Np // 128, 128), jnp.float32),
    )(pd)
    return out.reshape(Np)


def _mm_body(x_ref, w_ref, o_ref):
    o_ref[...] = jnp.dot(x_ref[...], w_ref[...],
                         preferred_element_type=jnp.float32)


def _mm(x, w):
    return pl.pallas_call(
        _mm_body,
        grid=(Np // _BM,),
        in_specs=[pl.BlockSpec((_BM, D), lambda i: (i, 0)),
                  pl.BlockSpec((D, H), lambda i: (0, 0))],
        out_specs=pl.BlockSpec((_BM, H), lambda i: (i, 0)),
        out_shape=jax.ShapeDtypeStruct((Np, H), jnp.float32),
    )(x, w)


def _cmb_mm_body(p_ref, b_ref, w_ref, o_ref):
    act = jnp.maximum(p_ref[0] + p_ref[1] + b_ref[...], 0.0)
    o_ref[...] = jnp.dot(act, w_ref[...], preferred_element_type=jnp.float32)


def _cmb_mm(p, b, w):
    return pl.pallas_call(
        _cmb_mm_body,
        grid=(Np // _BM,),
        in_specs=[pl.BlockSpec((NSC, _BM, H), lambda i: (0, i, 0)),
                  pl.BlockSpec((1, H), lambda i: (0, 0)),
                  pl.BlockSpec((H, H), lambda i: (0, 0))],
        out_specs=pl.BlockSpec((_BM, H), lambda i: (i, 0)),
        out_shape=jax.ShapeDtypeStruct((Np, H), jnp.float32),
    )(p, b.reshape(1, H), w)


def _cmb_final_body(p_ref, b_ref, o_ref):
    o_ref[...] = jnp.maximum(p_ref[0] + p_ref[1] + b_ref[...], 0.0)


def _cmb_final(p, b):
    return pl.pallas_call(
        _cmb_final_body,
        grid=(Np // _BM,),
        in_specs=[pl.BlockSpec((NSC, _BM, H), lambda i: (0, i, 0)),
                  pl.BlockSpec((1, H), lambda i: (0, 0))],
        out_specs=pl.BlockSpec((_BM, H), lambda i: (i, 0)),
        out_shape=jax.ShapeDtypeStruct((Np, H), jnp.float32),
    )(p, b.reshape(1, H))


def kernel(x, edge_index, edge_attr, W1, b1, W2, b2, W3, b3):
    row, col, ew = edge_index[0], edge_index[1], edge_attr
    loop = jnp.arange(Np, dtype=jnp.int32)
    pad = L - (E + Np)
    ext_row = jnp.concatenate(
        [row, loop, jnp.zeros((pad,), jnp.int32)]).reshape(SLABS, CH, B)
    ext_col = jnp.concatenate(
        [col, loop, jnp.full((pad,), Np - 1, jnp.int32)]).reshape(SLABS, CH, B)
    ext_ew = jnp.concatenate(
        [ew, jnp.ones((Np,), jnp.float32),
         jnp.zeros((pad,), jnp.float32)]).reshape(SLABS, CH, B)
    xp = jnp.pad(x, ((0, Np - N), (0, 0)))

    pdeg = _deg(ext_col, ext_ew)
    dis = _dis(pdeg)
    prow, pcol, pnorm, cnt = _part(ext_row, ext_col, ext_ew, dis)

    h = _mm(xp, W1)
    p = _agg(h, prow, pcol, pnorm, cnt)
    h = _cmb_mm(p, b1, W2)
    p = _agg(h, prow, pcol, pnorm, cnt)
    h = _cmb_mm(p, b2, W3)
    p = _agg(h, prow, pcol, pnorm, cnt)
    y = _cmb_final(p, b3)
    return y[:N]


# speedup vs baseline: 1.1315x; 1.1315x over previous
import functools

import jax
import jax.numpy as jnp
from jax import lax
from jax.experimental import pallas as pl
from jax.experimental.pallas import tpu as pltpu
from jax.experimental.pallas import tpu_sc as plsc

N, E, D, H = 10000, 320000, 128, 128
Np = 10240
NSC, NTILE = 2, 16
SLABS = NSC * NTILE
CH, B = 88, 128
KB = 8
NB = CH // KB
WIN = 2048
WSH = 11
NBK = Np // WIN
CAPC = CH
CAPW = NBK * CAPC * B
SCAP = CAPC * B + 128
L = SLABS * CH * B
RPT = Np // NTILE

_MESH = plsc.VectorSubcoreMesh(
    core_axis_name="c", subcore_axis_name="s",
    num_cores=NSC, num_subcores=NTILE)


def _deg_body(col_hbm, ew_hbm, out_hbm, colv, ewv, bounce, acc):
    c = lax.axis_index("c")
    s = lax.axis_index("s")
    w = c * NTILE + s

    def zb(i, carry):
        bounce[pl.ds(i * 16, 16)] = jnp.zeros((16,), jnp.float32)
        return carry
    lax.fori_loop(0, RPT // 16, zb, 0)
    pltpu.sync_copy(bounce, acc.at[pl.ds(s * RPT, RPT)])
    plsc.subcore_barrier()

    def blk_body(blk, carry):
        pltpu.sync_copy(col_hbm.at[w, pl.ds(blk * KB, KB)], colv)
        pltpu.sync_copy(ew_hbm.at[w, pl.ds(blk * KB, KB)], ewv)

        def body(i, carry2):
            pltpu.sync_copy(ewv.at[i], acc.at[colv.at[i]], add=True)
            return carry2
        lax.fori_loop(0, KB, body, 0)
        return carry
    lax.fori_loop(0, NB, blk_body, 0)
    plsc.subcore_barrier()

    pltpu.sync_copy(acc.at[pl.ds(s * RPT, RPT)], bounce)
    pltpu.sync_copy(bounce, out_hbm.at[c, pl.ds(s * RPT, RPT)])


_deg = functools.partial(
    pl.kernel,
    out_type=jax.ShapeDtypeStruct((NSC, Np), jnp.float32),
    mesh=_MESH,
    compiler_params=pltpu.CompilerParams(needs_layout_passes=False),
    scratch_types=[
        pltpu.VMEM((KB, B), jnp.int32),
        pltpu.VMEM((KB, B), jnp.float32),
        pltpu.VMEM((RPT,), jnp.float32),
        pltpu.VMEM_SHARED((Np,), jnp.float32),
    ],
)(_deg_body)


def _part_body(row_hbm, col_hbm, ew_hbm, dis_hbm,
               prow_hbm, pcol_hbm, pnorm_hbm, cnt_hbm,
               rowv, colv, ewv, srow, scol, snorm, disv, cntv):
    c = lax.axis_index("c")
    s = lax.axis_index("s")
    w = c * NTILE + s
    pltpu.sync_copy(dis_hbm, disv)

    zi = jnp.zeros((16,), jnp.int32)
    zc = jnp.full((16,), Np - 1, jnp.int32)
    zf = jnp.zeros((16,), jnp.float32)

    def prefill(n16, _):
        def pf(i, carry):
            sl = pl.ds(i * 16, 16)
            srow[sl] = zi
            scol[sl] = zc
            snorm[sl] = zf
            return carry
        lax.fori_loop(0, n16, pf, 0)

    prefill(SCAP // 16, None)
    nbs = []
    for b in range(NBK):
        def blk(kb, off):
            sl_blk = pl.ds(kb * KB, KB)
            pltpu.sync_copy(row_hbm.at[w, sl_blk], rowv)
            pltpu.sync_copy(col_hbm.at[w, sl_blk], colv)
            pltpu.sync_copy(ew_hbm.at[w, sl_blk], ewv)

            def ch(i, off2):
                for j in range(B // 16):
                    sl = pl.ds(j * 16, 16)
                    r16 = rowv[i, sl]
                    c16 = colv[i, sl]
                    e16 = ewv[i, sl]
                    dr = plsc.load_gather(disv, [r16])
                    dc = plsc.load_gather(disv, [c16])
                    nrm = dr * e16 * dc
                    rel = r16 - b * WIN
                    msk = lax.shift_right_logical(r16, WSH) == b
                    plsc.store_compressed(srow.at[pl.ds(off2, 16)], rel, mask=msk)
                    plsc.store_compressed(scol.at[pl.ds(off2, 16)], c16, mask=msk)
                    plsc.store_compressed(snorm.at[pl.ds(off2, 16)], nrm, mask=msk)
                    pc = plsc.all_reduce_population_count(msk)
                    off2 = off2 + pc[0]
                return off2
            return lax.fori_loop(0, KB, ch, off)
        off = lax.fori_loop(0, NB, blk, 0)
        fl = pl.ds(b * CAPC * B, CAPC * B)
        pltpu.sync_copy(srow.at[pl.ds(0, CAPC * B)], prow_hbm.at[w, fl])
        pltpu.sync_copy(scol.at[pl.ds(0, CAPC * B)], pcol_hbm.at[w, fl])
        pltpu.sync_copy(snorm.at[pl.ds(0, CAPC * B)], pnorm_hbm.at[w, fl])
        nbs.append((off + B - 1) // B)
        prefill((off + 15) // 16, None)

    lanes = lax.iota(jnp.int32, 16)
    cv = jnp.zeros((16,), jnp.int32)
    for b in range(NBK):
        cv = jnp.where(lanes == b, jnp.full((16,), nbs[b]), cv)
    z16 = jnp.zeros((16,), jnp.int32)
    for q in range(8):
        cntv[pl.ds(q * 16, 16)] = z16
    cntv[pl.ds(0, 16)] = cv
    pltpu.sync_copy(cntv, cnt_hbm.at[w])


_part = functools.partial(
    pl.kernel,
    out_type=(jax.ShapeDtypeStruct((SLABS, CAPW), jnp.int32),
              jax.ShapeDtypeStruct((SLABS, CAPW), jnp.int32),
              jax.ShapeDtypeStruct((SLABS, CAPW), jnp.float32),
              jax.ShapeDtypeStruct((SLABS, 128), jnp.int32)),
    mesh=_MESH,
    compiler_params=pltpu.CompilerParams(needs_layout_passes=False),
    scratch_types=[
        pltpu.VMEM((KB, B), jnp.int32),
        pltpu.VMEM((KB, B), jnp.int32),
        pltpu.VMEM((KB, B), jnp.float32),
        pltpu.VMEM((SCAP,), jnp.int32),
        pltpu.VMEM((SCAP,), jnp.int32),
        pltpu.VMEM((SCAP,), jnp.float32),
        pltpu.VMEM((Np,), jnp.float32),
        pltpu.VMEM((128,), jnp.int32),
    ],
)(_part_body)


def _agg_body(h_hbm, prow_hbm, pcol_hbm, pnorm_hbm, cnt_hbm, out_hbm,
              rowf, colf, normf, rowv2, colv2, buf, cntv, win, acc,
              gsa, gsb, ssa, ssb):
    c = lax.axis_index("c")
    s = lax.axis_index("s")
    w = c * NTILE + s
    pltpu.sync_copy(cnt_hbm.at[w], cntv)

    def zb(r, carry):
        for j in range(B // 16):
            buf[0, r % 64, pl.ds(j * 16, 16)] = jnp.zeros((16,), jnp.float32)
            buf[1, r % 64, pl.ds(j * 16, 16)] = jnp.zeros((16,), jnp.float32)
        return carry
    lax.fori_loop(0, 64, zb, 0)
    for k in range(RPT // 64):
        pltpu.sync_copy(buf.at[k % 2], acc.at[pl.ds(s * RPT + k * 64, 64)])

    def scale(slot, nbase):
        def sc_body(rg, carry3):
            nv16 = normf[pl.ds(nbase + rg * 16, 16)]
            for r in range(16):
                nv = lax.gather(
                    nv16, jnp.full((16, 1), r, jnp.int32),
                    lax.GatherDimensionNumbers(
                        offset_dims=(), collapsed_slice_dims=(0,),
                        start_index_map=(0,)),
                    (1,), mode=lax.GatherScatterMode.PROMISE_IN_BOUNDS)
                row = rg * 16 + r
                for j in range(B // 16):
                    sl = pl.ds(j * 16, 16)
                    buf[slot, row, sl] = buf[slot, row, sl] * nv
            return carry3
        lax.fori_loop(0, 4, sc_body, 0)

    def gwait(sem):
        pltpu.make_async_copy(win.at[rowv2.at[0]], buf.at[0], sem).wait()

    def swait(sem):
        pltpu.make_async_copy(buf.at[0], acc.at[colv2.at[0]], sem).wait()

    for b in range(NBK):
        plsc.subcore_barrier()
        pltpu.sync_copy(h_hbm.at[pl.ds(b * WIN + s * B, B)],
                        win.at[pl.ds(s * B, B)])
        plsc.subcore_barrier()

        cv = cntv[pl.ds(0, 16)]
        nch = cv[b]
        nblk = (nch + KB - 1) // KB

        def blk(kb, carry):
            off = (b * CAPC + kb * KB) * B
            pltpu.sync_copy(prow_hbm.at[w, pl.ds(off, KB * B)], rowf)
            pltpu.sync_copy(pcol_hbm.at[w, pl.ds(off, KB * B)], colf)
            pltpu.sync_copy(pnorm_hbm.at[w, pl.ds(off, KB * B)], normf)

            def rsh(q, carry2):
                for k2 in range(4):
                    sl = pl.ds(k2 * 16, 16)
                    fsl = pl.ds(q * 64 + k2 * 16, 16)
                    rowv2[q, sl] = rowf[fsl]
                    colv2[q, sl] = colf[fsl]
                return carry2
            lax.fori_loop(0, 2 * KB, rsh, 0)

            nin = jnp.minimum(nch - kb * KB, KB)
            nsub = 2 * nin
            pltpu.async_copy(win.at[rowv2.at[0]], buf.at[0], gsa)

            def pair(k, carry2):
                s0 = 2 * k
                s1 = s0 + 1
                gwait(gsa)
                scale(0, s0 * 64)

                @pl.when(k > 0)
                def _():
                    swait(ssb)
                pltpu.async_copy(win.at[rowv2.at[s1]], buf.at[1], gsb)
                pltpu.async_copy(buf.at[0], acc.at[colv2.at[s0]], ssa,
                                add=True)
                gwait(gsb)
                scale(1, s1 * 64)
                swait(ssa)
                su2 = jnp.minimum(s0 + 2, nsub - 1)
                pltpu.async_copy(win.at[rowv2.at[su2]], buf.at[0], gsa)
                pltpu.async_copy(buf.at[1], acc.at[colv2.at[s1]], ssb,
                                add=True)
                return carry2
            lax.fori_loop(0, nin, pair, 0)
            gwait(gsa)
            swait(ssb)
            return carry
        lax.fori_loop(0, nblk, blk, 0)
    plsc.subcore_barrier()

    for k in range(RPT // 64):
        rs = s * RPT + k * 64
        pltpu.sync_copy(acc.at[pl.ds(rs, 64)], buf.at[0])
        pltpu.sync_copy(buf.at[0], out_hbm.at[c, pl.ds(rs, 64)])


_agg = functools.partial(
    pl.kernel,
    out_type=jax.ShapeDtypeStruct((NSC, Np, H), jnp.float32),
    mesh=_MESH,
    compiler_params=pltpu.CompilerParams(needs_layout_passes=False),
    scratch_types=[
        pltpu.VMEM((KB * B,), jnp.int32),
        pltpu.VMEM((KB * B,), jnp.int32),
        pltpu.VMEM((KB * B,), jnp.float32),
        pltpu.VMEM((2 * KB, 64), jnp.int32),
        pltpu.VMEM((2 * KB, 64), jnp.int32),
        pltpu.VMEM((2, 64, H), jnp.float32),
        pltpu.VMEM((128,), jnp.int32),
        pltpu.VMEM_SHARED((WIN, H), jnp.float32),
        pltpu.VMEM_SHARED((Np, H), jnp.float32),
        pltpu.SemaphoreType.DMA,
        pltpu.SemaphoreType.DMA,
        pltpu.SemaphoreType.DMA,
        pltpu.SemaphoreType.DMA,
    ],
)(_agg_body)


_BM = 512


def _dis_body(pd_ref, o_ref):
    deg = pd_ref[0] + pd_ref[1]
    o_ref[...] = lax.rsqrt(jnp.maximum(deg, 1e-12))


def _dis(pdeg):
    pd = pdeg.reshape(NSC, Np // 128, 128)
    out = pl.pallas_call(
        _dis_body,
        out_shape=jax.ShapeDtypeStruct((Np // 128, 128), jnp.float32),
    )(pd)
    return out.reshape(Np)


def _mm_body(x_ref, w_ref, o_ref):
    o_ref[...] = jnp.dot(x_ref[...], w_ref[...],
                         preferred_element_type=jnp.float32)


def _mm(x, w):
    return pl.pallas_call(
        _mm_body,
        grid=(Np // _BM,),
        in_specs=[pl.BlockSpec((_BM, D), lambda i: (i, 0)),
                  pl.BlockSpec((D, H), lambda i: (0, 0))],
        out_specs=pl.BlockSpec((_BM, H), lambda i: (i, 0)),
        out_shape=jax.ShapeDtypeStruct((Np, H), jnp.float32),
    )(x, w)


def _cmb_mm_body(p_ref, b_ref, w_ref, o_ref):
    act = jnp.maximum(p_ref[0] + p_ref[1] + b_ref[...], 0.0)
    o_ref[...] = jnp.dot(act, w_ref[...], preferred_element_type=jnp.float32)


def _cmb_mm(p, b, w):
    return pl.pallas_call(
        _cmb_mm_body,
        grid=(Np // _BM,),
        in_specs=[pl.BlockSpec((NSC, _BM, H), lambda i: (0, i, 0)),
                  pl.BlockSpec((1, H), lambda i: (0, 0)),
                  pl.BlockSpec((H, H), lambda i: (0, 0))],
        out_specs=pl.BlockSpec((_BM, H), lambda i: (i, 0)),
        out_shape=jax.ShapeDtypeStruct((Np, H), jnp.float32),
    )(p, b.reshape(1, H), w)


def _cmb_final_body(p_ref, b_ref, o_ref):
    o_ref[...] = jnp.maximum(p_ref[0] + p_ref[1] + b_ref[...], 0.0)


def _cmb_final(p, b):
    return pl.pallas_call(
        _cmb_final_body,
        grid=(Np // _BM,),
        in_specs=[pl.BlockSpec((NSC, _BM, H), lambda i: (0, i, 0)),
                  pl.BlockSpec((1, H), lambda i: (0, 0))],
        out_specs=pl.BlockSpec((_BM, H), lambda i: (i, 0)),
        out_shape=jax.ShapeDtypeStruct((Np, H), jnp.float32),
    )(p, b.reshape(1, H))


def kernel(x, edge_index, edge_attr, W1, b1, W2, b2, W3, b3):
    row, col, ew = edge_index[0], edge_index[1], edge_attr
    loop = jnp.arange(Np, dtype=jnp.int32)
    pad = L - (E + Np)
    ext_row = jnp.concatenate(
        [row, loop, jnp.zeros((pad,), jnp.int32)]).reshape(SLABS, CH, B)
    ext_col = jnp.concatenate(
        [col, loop, jnp.full((pad,), Np - 1, jnp.int32)]).reshape(SLABS, CH, B)
    ext_ew = jnp.concatenate(
        [ew, jnp.ones((Np,), jnp.float32),
         jnp.zeros((pad,), jnp.float32)]).reshape(SLABS, CH, B)
    xp = jnp.pad(x, ((0, Np - N), (0, 0)))

    pdeg = _deg(ext_col, ext_ew)
    dis = _dis(pdeg)
    prow, pcol, pnorm, cnt = _part(ext_row, ext_col, ext_ew, dis)

    h = _mm(xp, W1)
    p = _agg(h, prow, pcol, pnorm, cnt)
    h = _cmb_mm(p, b1, W2)
    p = _agg(h, prow, pcol, pnorm, cnt)
    h = _cmb_mm(p, b2, W3)
    p = _agg(h, prow, pcol, pnorm, cnt)
    y = _cmb_final(p, b3)
    return y[:N]


# speedup vs baseline: 1.2278x; 1.0852x over previous
import functools

import jax
import jax.numpy as jnp
from jax import lax
from jax.experimental import pallas as pl
from jax.experimental.pallas import tpu as pltpu
from jax.experimental.pallas import tpu_sc as plsc

N, E, D, H = 10000, 320000, 128, 128
Np = 10240
NSC, NTILE = 2, 16
SLABS = NSC * NTILE
CH, B = 88, 128
KB = 8
NB = CH // KB
WIN = 2048
WSH = 11
NBK = Np // WIN
CAPC = CH
KBA = 16
CAPW = NBK * CAPC * B + KBA * B
SCAP = CAPC * B + 128
L = SLABS * CH * B
RPT = Np // NTILE

_MESH = plsc.VectorSubcoreMesh(
    core_axis_name="c", subcore_axis_name="s",
    num_cores=NSC, num_subcores=NTILE)


def _deg_body(col_hbm, ew_hbm, out_hbm, colv, ewv, bounce, acc):
    c = lax.axis_index("c")
    s = lax.axis_index("s")
    w = c * NTILE + s

    def zb(i, carry):
        bounce[pl.ds(i * 16, 16)] = jnp.zeros((16,), jnp.float32)
        return carry
    lax.fori_loop(0, RPT // 16, zb, 0)
    pltpu.sync_copy(bounce, acc.at[pl.ds(s * RPT, RPT)])
    plsc.subcore_barrier()

    def blk_body(blk, carry):
        pltpu.sync_copy(col_hbm.at[w, pl.ds(blk * KB, KB)], colv)
        pltpu.sync_copy(ew_hbm.at[w, pl.ds(blk * KB, KB)], ewv)

        def body(i, carry2):
            pltpu.sync_copy(ewv.at[i], acc.at[colv.at[i]], add=True)
            return carry2
        lax.fori_loop(0, KB, body, 0)
        return carry
    lax.fori_loop(0, NB, blk_body, 0)
    plsc.subcore_barrier()

    pltpu.sync_copy(acc.at[pl.ds(s * RPT, RPT)], bounce)
    pltpu.sync_copy(bounce, out_hbm.at[c, pl.ds(s * RPT, RPT)])


_deg = functools.partial(
    pl.kernel,
    out_type=jax.ShapeDtypeStruct((NSC, Np), jnp.float32),
    mesh=_MESH,
    compiler_params=pltpu.CompilerParams(needs_layout_passes=False),
    scratch_types=[
        pltpu.VMEM((KB, B), jnp.int32),
        pltpu.VMEM((KB, B), jnp.float32),
        pltpu.VMEM((RPT,), jnp.float32),
        pltpu.VMEM_SHARED((Np,), jnp.float32),
    ],
)(_deg_body)


def _part_body(row_hbm, col_hbm, ew_hbm, dis_hbm,
               prow_hbm, pcol_hbm, pnorm_hbm, cnt_hbm,
               rowv, colv, ewv, rowS, colS, normS,
               srow, scol, snorm, disv, cntv):
    c = lax.axis_index("c")
    s = lax.axis_index("s")
    w = c * NTILE + s
    pltpu.sync_copy(dis_hbm, disv)

    def blk0(kb, carry):
        sl_blk = pl.ds(kb * KB, KB)
        pltpu.sync_copy(row_hbm.at[w, sl_blk], rowv)
        pltpu.sync_copy(col_hbm.at[w, sl_blk], colv)
        pltpu.sync_copy(ew_hbm.at[w, sl_blk], ewv)

        def ch(i, carry2):
            g = kb * KB + i
            for j in range(B // 16):
                sl = pl.ds(j * 16, 16)
                r16 = rowv[i, sl]
                c16 = colv[i, sl]
                e16 = ewv[i, sl]
                dr = plsc.load_gather(disv, [r16])
                dc = plsc.load_gather(disv, [c16])
                rowS[g, sl] = r16
                colS[g, sl] = c16
                normS[g, sl] = dr * e16 * dc
            return carry2
        lax.fori_loop(0, KB, ch, 0)
        return carry
    lax.fori_loop(0, NB, blk0, 0)

    zi = jnp.zeros((16,), jnp.int32)
    zc = jnp.full((16,), Np - 1, jnp.int32)
    zf = jnp.zeros((16,), jnp.float32)

    def prefill(n16, _):
        def pf(i, carry):
            sl = pl.ds(i * 16, 16)
            srow[sl] = zi
            scol[sl] = zc
            snorm[sl] = zf
            return carry
        lax.fori_loop(0, n16, pf, 0)

    prefill(SCAP // 16, None)
    nbs = []
    for b in range(NBK):
        def scan(g, off):
            def ch(j_off):
                return j_off

            off2 = off
            for j in range(B // 16):
                sl = pl.ds(j * 16, 16)
                r16 = rowS[g, sl]
                c16 = colS[g, sl]
                n16 = normS[g, sl]
                rel = r16 - b * WIN
                msk = lax.shift_right_logical(r16, WSH) == b
                plsc.store_compressed(srow.at[pl.ds(off2, 16)], rel, mask=msk)
                plsc.store_compressed(scol.at[pl.ds(off2, 16)], c16, mask=msk)
                plsc.store_compressed(snorm.at[pl.ds(off2, 16)], n16, mask=msk)
                pc = plsc.all_reduce_population_count(msk)
                off2 = off2 + pc[0]
            return off2
        off = lax.fori_loop(0, CH, scan, 0)
        fl = pl.ds(b * CAPC * B, CAPC * B)
        pltpu.sync_copy(srow.at[pl.ds(0, CAPC * B)], prow_hbm.at[w, fl])
        pltpu.sync_copy(scol.at[pl.ds(0, CAPC * B)], pcol_hbm.at[w, fl])
        pltpu.sync_copy(snorm.at[pl.ds(0, CAPC * B)], pnorm_hbm.at[w, fl])
        nbs.append((off + B - 1) // B)
        prefill((off + 15) // 16, None)

    lanes = lax.iota(jnp.int32, 16)
    cv = jnp.zeros((16,), jnp.int32)
    for b in range(NBK):
        cv = jnp.where(lanes == b, jnp.full((16,), nbs[b]), cv)
    z16 = jnp.zeros((16,), jnp.int32)
    for q in range(8):
        cntv[pl.ds(q * 16, 16)] = z16
    cntv[pl.ds(0, 16)] = cv
    pltpu.sync_copy(cntv, cnt_hbm.at[w])


_part = functools.partial(
    pl.kernel,
    out_type=(jax.ShapeDtypeStruct((SLABS, CAPW), jnp.int32),
              jax.ShapeDtypeStruct((SLABS, CAPW), jnp.int32),
              jax.ShapeDtypeStruct((SLABS, CAPW), jnp.float32),
              jax.ShapeDtypeStruct((SLABS, 128), jnp.int32)),
    mesh=_MESH,
    compiler_params=pltpu.CompilerParams(needs_layout_passes=False),
    scratch_types=[
        pltpu.VMEM((KB, B), jnp.int32),
        pltpu.VMEM((KB, B), jnp.int32),
        pltpu.VMEM((KB, B), jnp.float32),
        pltpu.VMEM((CH, B), jnp.int32),
        pltpu.VMEM((CH, B), jnp.int32),
        pltpu.VMEM((CH, B), jnp.float32),
        pltpu.VMEM((SCAP,), jnp.int32),
        pltpu.VMEM((SCAP,), jnp.int32),
        pltpu.VMEM((SCAP,), jnp.float32),
        pltpu.VMEM((Np,), jnp.float32),
        pltpu.VMEM((128,), jnp.int32),
    ],
)(_part_body)


def _agg_body(h_hbm, prow_hbm, pcol_hbm, pnorm_hbm, cnt_hbm, out_hbm,
              rowf, colf, normf, rowv2, colv2, buf, cntv, win, acc,
              gsa, gsb, ssa, ssb):
    c = lax.axis_index("c")
    s = lax.axis_index("s")
    w = c * NTILE + s
    pltpu.sync_copy(cnt_hbm.at[w], cntv)

    def zb(r, carry):
        for j in range(B // 16):
            buf[0, r % 64, pl.ds(j * 16, 16)] = jnp.zeros((16,), jnp.float32)
            buf[1, r % 64, pl.ds(j * 16, 16)] = jnp.zeros((16,), jnp.float32)
        return carry
    lax.fori_loop(0, 64, zb, 0)
    for k in range(RPT // 64):
        pltpu.sync_copy(buf.at[k % 2], acc.at[pl.ds(s * RPT + k * 64, 64)])

    def scale(slot, nbase):
        def sc_body(rg, carry3):
            nv16 = normf[pl.ds(nbase + rg * 16, 16)]
            for r in range(16):
                nv = lax.gather(
                    nv16, jnp.full((16, 1), r, jnp.int32),
                    lax.GatherDimensionNumbers(
                        offset_dims=(), collapsed_slice_dims=(0,),
                        start_index_map=(0,)),
                    (1,), mode=lax.GatherScatterMode.PROMISE_IN_BOUNDS)
                row = rg * 16 + r
                for j in range(B // 16):
                    sl = pl.ds(j * 16, 16)
                    buf[slot, row, sl] = buf[slot, row, sl] * nv
            return carry3
        lax.fori_loop(0, 4, sc_body, 0)

    def gwait(sem):
        pltpu.make_async_copy(win.at[rowv2.at[0]], buf.at[0], sem).wait()

    def swait(sem):
        pltpu.make_async_copy(buf.at[0], acc.at[colv2.at[0]], sem).wait()

    for b in range(NBK):
        plsc.subcore_barrier()
        pltpu.sync_copy(h_hbm.at[pl.ds(b * WIN + s * B, B)],
                        win.at[pl.ds(s * B, B)])
        plsc.subcore_barrier()

        cv = cntv[pl.ds(0, 16)]
        nch = cv[b]
        nblk = (nch + KBA - 1) // KBA

        def blk(kb, carry):
            off = (b * CAPC + kb * KBA) * B
            pltpu.sync_copy(prow_hbm.at[w, pl.ds(off, KBA * B)], rowf)
            pltpu.sync_copy(pcol_hbm.at[w, pl.ds(off, KBA * B)], colf)
            pltpu.sync_copy(pnorm_hbm.at[w, pl.ds(off, KBA * B)], normf)

            def rsh(q, carry2):
                for k2 in range(4):
                    sl = pl.ds(k2 * 16, 16)
                    fsl = pl.ds(q * 64 + k2 * 16, 16)
                    rowv2[q, sl] = rowf[fsl]
                    colv2[q, sl] = colf[fsl]
                return carry2
            lax.fori_loop(0, 2 * KBA, rsh, 0)

            nin = jnp.minimum(nch - kb * KBA, KBA)
            nsub = 2 * nin
            pltpu.async_copy(win.at[rowv2.at[0]], buf.at[0], gsa)

            def pair(k, carry2):
                s0 = 2 * k
                s1 = s0 + 1
                gwait(gsa)
                scale(0, s0 * 64)

                @pl.when(k > 0)
                def _():
                    swait(ssb)
                pltpu.async_copy(win.at[rowv2.at[s1]], buf.at[1], gsb)
                pltpu.async_copy(buf.at[0], acc.at[colv2.at[s0]], ssa,
                                add=True)
                gwait(gsb)
                scale(1, s1 * 64)
                swait(ssa)
                su2 = jnp.minimum(s0 + 2, nsub - 1)
                pltpu.async_copy(win.at[rowv2.at[su2]], buf.at[0], gsa)
                pltpu.async_copy(buf.at[1], acc.at[colv2.at[s1]], ssb,
                                add=True)
                return carry2
            lax.fori_loop(0, nin, pair, 0)
            gwait(gsa)
            swait(ssb)
            return carry
        lax.fori_loop(0, nblk, blk, 0)
    plsc.subcore_barrier()

    for k in range(RPT // 64):
        rs = s * RPT + k * 64
        pltpu.sync_copy(acc.at[pl.ds(rs, 64)], buf.at[0])
        pltpu.sync_copy(buf.at[0], out_hbm.at[c, pl.ds(rs, 64)])


_agg = functools.partial(
    pl.kernel,
    out_type=jax.ShapeDtypeStruct((NSC, Np, H), jnp.float32),
    mesh=_MESH,
    compiler_params=pltpu.CompilerParams(needs_layout_passes=False),
    scratch_types=[
        pltpu.VMEM((KBA * B,), jnp.int32),
        pltpu.VMEM((KBA * B,), jnp.int32),
        pltpu.VMEM((KBA * B,), jnp.float32),
        pltpu.VMEM((2 * KBA, 64), jnp.int32),
        pltpu.VMEM((2 * KBA, 64), jnp.int32),
        pltpu.VMEM((2, 64, H), jnp.float32),
        pltpu.VMEM((128,), jnp.int32),
        pltpu.VMEM_SHARED((WIN, H), jnp.float32),
        pltpu.VMEM_SHARED((Np, H), jnp.float32),
        pltpu.SemaphoreType.DMA,
        pltpu.SemaphoreType.DMA,
        pltpu.SemaphoreType.DMA,
        pltpu.SemaphoreType.DMA,
    ],
)(_agg_body)


_BM = 512


def _dis_body(pd_ref, o_ref):
    deg = pd_ref[0] + pd_ref[1]
    o_ref[...] = lax.rsqrt(jnp.maximum(deg, 1e-12))


def _dis(pdeg):
    pd = pdeg.reshape(NSC, Np // 128, 128)
    out = pl.pallas_call(
        _dis_body,
        out_shape=jax.ShapeDtypeStruct((Np // 128, 128), jnp.float32),
    )(pd)
    return out.reshape(Np)


def _mm_body(x_ref, w_ref, o_ref):
    o_ref[...] = jnp.dot(x_ref[...], w_ref[...],
                         preferred_element_type=jnp.float32)


def _mm(x, w):
    return pl.pallas_call(
        _mm_body,
        grid=(Np // _BM,),
        in_specs=[pl.BlockSpec((_BM, D), lambda i: (i, 0)),
                  pl.BlockSpec((D, H), lambda i: (0, 0))],
        out_specs=pl.BlockSpec((_BM, H), lambda i: (i, 0)),
        out_shape=jax.ShapeDtypeStruct((Np, H), jnp.float32),
    )(x, w)


def _cmb_mm_body(p_ref, b_ref, w_ref, o_ref):
    act = jnp.maximum(p_ref[0] + p_ref[1] + b_ref[...], 0.0)
    o_ref[...] = jnp.dot(act, w_ref[...], preferred_element_type=jnp.float32)


def _cmb_mm(p, b, w):
    return pl.pallas_call(
        _cmb_mm_body,
        grid=(Np // _BM,),
        in_specs=[pl.BlockSpec((NSC, _BM, H), lambda i: (0, i, 0)),
                  pl.BlockSpec((1, H), lambda i: (0, 0)),
                  pl.BlockSpec((H, H), lambda i: (0, 0))],
        out_specs=pl.BlockSpec((_BM, H), lambda i: (i, 0)),
        out_shape=jax.ShapeDtypeStruct((Np, H), jnp.float32),
    )(p, b.reshape(1, H), w)


def _cmb_final_body(p_ref, b_ref, o_ref):
    o_ref[...] = jnp.maximum(p_ref[0] + p_ref[1] + b_ref[...], 0.0)


def _cmb_final(p, b):
    return pl.pallas_call(
        _cmb_final_body,
        grid=(Np // _BM,),
        in_specs=[pl.BlockSpec((NSC, _BM, H), lambda i: (0, i, 0)),
                  pl.BlockSpec((1, H), lambda i: (0, 0))],
        out_specs=pl.BlockSpec((_BM, H), lambda i: (i, 0)),
        out_shape=jax.ShapeDtypeStruct((Np, H), jnp.float32),
    )(p, b.reshape(1, H))


def kernel(x, edge_index, edge_attr, W1, b1, W2, b2, W3, b3):
    row, col, ew = edge_index[0], edge_index[1], edge_attr
    loop = jnp.arange(Np, dtype=jnp.int32)
    pad = L - (E + Np)
    ext_row = jnp.concatenate(
        [row, loop, jnp.zeros((pad,), jnp.int32)]).reshape(SLABS, CH, B)
    ext_col = jnp.concatenate(
        [col, loop, jnp.full((pad,), Np - 1, jnp.int32)]).reshape(SLABS, CH, B)
    ext_ew = jnp.concatenate(
        [ew, jnp.ones((Np,), jnp.float32),
         jnp.zeros((pad,), jnp.float32)]).reshape(SLABS, CH, B)
    xp = jnp.pad(x, ((0, Np - N), (0, 0)))

    pdeg = _deg(ext_col, ext_ew)
    dis = _dis(pdeg)
    prow, pcol, pnorm, cnt = _part(ext_row, ext_col, ext_ew, dis)

    h = _mm(xp, W1)
    p = _agg(h, prow, pcol, pnorm, cnt)
    h = _cmb_mm(p, b1, W2)
    p = _agg(h, prow, pcol, pnorm, cnt)
    h = _cmb_mm(p, b2, W3)
    p = _agg(h, prow, pcol, pnorm, cnt)
    y = _cmb_final(p, b3)
    return y[:N]
